# Initial kernel scaffold; baseline (speedup 1.0000x reference)
import jax
import jax.numpy as jnp
from jax.experimental import pallas as pl


def kernel(x, embed_table, proj_w, proj_b):
    raise NotImplementedError("write your pallas kernel here")


# speedup vs baseline: 1.2093x; 1.2093x over previous
import functools

import jax
import jax.numpy as jnp
from jax import lax
from jax.experimental import pallas as pl
from jax.experimental.pallas import tpu as pltpu
from jax.experimental.pallas import tpu_sc as plsc

_NUM_CORES = 2
_NUM_SUBCORES = 16
_NUM_WORKERS = _NUM_CORES * _NUM_SUBCORES

_GATHER = 128
_CHUNK = 8


def _project_body(emb_ref, w_ref, b_ref, out_ref):
    out_ref[...] = lax.dot_general(
        emb_ref[...], w_ref[...],
        dimension_numbers=(((1,), (1,)), ((), ())),
        preferred_element_type=jnp.float32,
    ) + b_ref[...]


def _project_table(embed_table, proj_w, proj_b):
    v, d = embed_table.shape
    return pl.pallas_call(
        _project_body,
        out_shape=jax.ShapeDtypeStruct((v, d), jnp.float32),
    )(embed_table, proj_w, proj_b.reshape(1, d))


def _sc_lookup(table_proj, idx_flat):
    b_total = idx_flat.shape[0] * idx_flat.shape[1]
    d = table_proj.shape[1]
    per_worker = b_total // _NUM_WORKERS
    n_gathers = per_worker // _GATHER
    n_chunks = n_gathers // _CHUNK

    mesh = plsc.VectorSubcoreMesh(
        core_axis_name="c", subcore_axis_name="s")

    @functools.partial(
        pl.kernel,
        out_type=jax.ShapeDtypeStruct((b_total, d), jnp.float32),
        mesh=mesh,
        scratch_types=[
            pltpu.VMEM((_CHUNK, _GATHER), jnp.int32),
            pltpu.VMEM((_CHUNK * _GATHER, d), jnp.float32),
            pltpu.SemaphoreType.DMA,
        ],
        compiler_params=pltpu.CompilerParams(use_tc_tiling_on_sc=False),
    )
    def lookup(table_hbm, idx_hbm, out_hbm, idx_v, rows_v, sem):
        wid = lax.axis_index("s") * _NUM_CORES + lax.axis_index("c")
        base = wid * per_worker

        def body(i, carry):
            off = pl.multiple_of(base + i * (_CHUNK * _GATHER),
                                 _CHUNK * _GATHER)
            row = pl.multiple_of((base // _GATHER) + i * _CHUNK, _CHUNK)
            pltpu.sync_copy(idx_hbm.at[pl.ds(row, _CHUNK)], idx_v)
            for j in range(_CHUNK):
                pltpu.async_copy(
                    table_hbm.at[idx_v.at[j]],
                    rows_v.at[pl.ds(j * _GATHER, _GATHER)],
                    sem,
                ).wait()
            pltpu.sync_copy(rows_v, out_hbm.at[pl.ds(off, _CHUNK * _GATHER)])
            return carry

        lax.fori_loop(0, n_chunks, body, 0)

    return lookup(table_proj, idx_flat)


def kernel(x, embed_table, proj_w, proj_b):
    bsz, seq = x.shape
    d = embed_table.shape[1]
    table_proj = _project_table(embed_table, proj_w, proj_b)
    idx_flat = x.reshape(-1, _GATHER).astype(jnp.int32)
    out_flat = _sc_lookup(table_proj, idx_flat)
    return out_flat.reshape(bsz, seq, d)


# speedup vs baseline: 1.2152x; 1.0048x over previous
import functools

import jax
import jax.numpy as jnp
from jax import lax
from jax.experimental import pallas as pl
from jax.experimental.pallas import tpu as pltpu
from jax.experimental.pallas import tpu_sc as plsc

_NUM_CORES = 2
_NUM_SUBCORES = 16
_NUM_WORKERS = _NUM_CORES * _NUM_SUBCORES

_GATHER = 128
_CHUNK = 4
_ROWS = _CHUNK * _GATHER


def _project_body(emb_ref, w_ref, b_ref, out_ref):
    out_ref[...] = lax.dot_general(
        emb_ref[...], w_ref[...],
        dimension_numbers=(((1,), (1,)), ((), ())),
        preferred_element_type=jnp.float32,
    ) + b_ref[...]


def _project_table(embed_table, proj_w, proj_b):
    v, d = embed_table.shape
    return pl.pallas_call(
        _project_body,
        out_shape=jax.ShapeDtypeStruct((v, d), jnp.float32),
    )(embed_table, proj_w, proj_b.reshape(1, d))


def _sc_lookup(table_proj, idx2d):
    b_total = idx2d.shape[0] * idx2d.shape[1]
    d = table_proj.shape[1]
    per_worker = b_total // _NUM_WORKERS
    n_chunks = per_worker // _ROWS
    half = n_chunks // 2

    mesh = plsc.VectorSubcoreMesh(
        core_axis_name="c", subcore_axis_name="s")

    @functools.partial(
        pl.kernel,
        out_type=jax.ShapeDtypeStruct((b_total, d), jnp.float32),
        mesh=mesh,
        scratch_types=[
            pltpu.VMEM((2, _CHUNK, _GATHER), jnp.int32),
            pltpu.VMEM((2, _ROWS, d), jnp.float32),
            [pltpu.SemaphoreType.DMA] * 2,
            [pltpu.SemaphoreType.DMA] * 2,
            [pltpu.SemaphoreType.DMA] * 2,
        ],
        compiler_params=pltpu.CompilerParams(use_tc_tiling_on_sc=False),
    )
    def lookup(table_hbm, idx_hbm, out_hbm, idx_v, rows_v, isem, gsem, osem):
        wid = lax.axis_index("s") * _NUM_CORES + lax.axis_index("c")
        base = wid * per_worker
        rbase = base // _GATHER

        def idx_copy(c, p):
            row = pl.multiple_of(rbase + c * _CHUNK, _CHUNK)
            return pltpu.make_async_copy(
                idx_hbm.at[pl.ds(row, _CHUNK)], idx_v.at[p], isem[p])

        def gather(c, p, j):
            del c
            return pltpu.make_async_copy(
                table_hbm.at[idx_v.at[p, j]],
                rows_v.at[p, pl.ds(j * _GATHER, _GATHER)],
                gsem[p])

        def write(c, p):
            off = pl.multiple_of(base + c * _ROWS, _ROWS)
            return pltpu.make_async_copy(
                rows_v.at[p], out_hbm.at[pl.ds(off, _ROWS)], osem[p])

        idx_copy(0, 0).start()
        idx_copy(1, 1).start()

        def retire(c, q, prefetch_pred):
            for j in range(_CHUNK):
                gather(c, q, j).wait()
            write(c, q).start()
            if prefetch_pred is True:
                idx_copy(c + 2, q).start()
            else:
                @pl.when(prefetch_pred)
                def _():
                    idx_copy(c + 2, q).start()

        def body(g, carry):
            for u in (0, 1):
                c = 2 * g + u
                p, q = u, 1 - u
                @pl.when(g >= 1)
                def _():
                    write(c - 2, p).wait()
                idx_copy(c, p).wait()
                for j in range(_CHUNK):
                    gather(c, p, j).start()
                if u == 0:
                    @pl.when(g >= 1)
                    def _():
                        retire(c - 1, q, True)
                else:
                    retire(c - 1, q, c + 2 < n_chunks)
            return carry

        lax.fori_loop(0, half, body, 0)

        last = n_chunks - 1
        for j in range(_CHUNK):
            gather(last, 1, j).wait()
        write(last, 1).start()
        write(last - 1, 0).wait()
        write(last, 1).wait()

    return lookup(table_proj, idx2d)


def kernel(x, embed_table, proj_w, proj_b):
    bsz, seq = x.shape
    d = embed_table.shape[1]
    table_proj = _project_table(embed_table, proj_w, proj_b)
    idx2d = x.reshape(-1, _GATHER).astype(jnp.int32)
    out_flat = _sc_lookup(table_proj, idx2d)
    return out_flat.reshape(bsz, seq, d)


# speedup vs baseline: 5.5885x; 4.5990x over previous
import functools

import jax
import jax.numpy as jnp
from jax import lax
from jax.experimental import pallas as pl
from jax.experimental.pallas import tpu as pltpu
from jax.experimental.pallas import tpu_sc as plsc

_NUM_CORES = 2
_NUM_SUBCORES = 16
_NUM_WORKERS = _NUM_CORES * _NUM_SUBCORES

_GATHER = 128
_CHUNK = 4
_ROWS = _CHUNK * _GATHER


def _project_body(emb_ref, w_ref, b_ref, out_ref):
    out_ref[...] = lax.dot_general(
        emb_ref[...], w_ref[...],
        dimension_numbers=(((1,), (1,)), ((), ())),
        preferred_element_type=jnp.float32,
    ) + b_ref[...]


def _project_table(embed_table, proj_w, proj_b):
    v, d = embed_table.shape
    return pl.pallas_call(
        _project_body,
        out_shape=jax.ShapeDtypeStruct((v, d), jnp.float32),
    )(embed_table, proj_w, proj_b.reshape(1, d))


def _sc_lookup(table_proj, idx2d):
    b_total = idx2d.shape[0] * idx2d.shape[1]
    d = table_proj.shape[1]
    per_worker = b_total // _NUM_WORKERS
    n_chunks = per_worker // _ROWS
    half = n_chunks // 2

    mesh = plsc.VectorSubcoreMesh(
        core_axis_name="c", subcore_axis_name="s")

    @functools.partial(
        pl.kernel,
        out_type=jax.ShapeDtypeStruct((b_total, d), jnp.float32),
        mesh=mesh,
        scratch_types=[
            pltpu.VMEM_SHARED((17, 64), jnp.float32),
            pltpu.VMEM((2, _CHUNK, _GATHER), jnp.int32),
            pltpu.VMEM((2, _ROWS, d), jnp.float32),
            [pltpu.SemaphoreType.DMA] * 2,
            [pltpu.SemaphoreType.DMA] * 2,
            [pltpu.SemaphoreType.DMA] * 2,
        ],
        compiler_params=pltpu.CompilerParams(use_tc_tiling_on_sc=False),
    )
    def lookup(table_hbm, idx_hbm, out_hbm, table_sh, idx_v, rows_v,
               isem, gsem, osem):
        sid = lax.axis_index("s")
        wid = sid * _NUM_CORES + lax.axis_index("c")
        base = wid * per_worker
        rbase = base // _GATHER

        @pl.when(sid == 0)
        def _():
            pltpu.sync_copy(table_hbm, table_sh)
        plsc.subcore_barrier()

        def idx_copy(c, p):
            row = pl.multiple_of(rbase + c * _CHUNK, _CHUNK)
            return pltpu.make_async_copy(
                idx_hbm.at[pl.ds(row, _CHUNK)], idx_v.at[p], isem[p])

        def gather(c, p, j):
            del c
            return pltpu.make_async_copy(
                table_sh.at[idx_v.at[p, j]],
                rows_v.at[p, pl.ds(j * _GATHER, _GATHER)],
                gsem[p])

        def write(c, p):
            off = pl.multiple_of(base + c * _ROWS, _ROWS)
            return pltpu.make_async_copy(
                rows_v.at[p], out_hbm.at[pl.ds(off, _ROWS)], osem[p])

        idx_copy(0, 0).start()
        idx_copy(1, 1).start()

        def retire(c, q, prefetch_pred):
            for j in range(_CHUNK):
                gather(c, q, j).wait()
            write(c, q).start()
            if prefetch_pred is True:
                idx_copy(c + 2, q).start()
            else:
                @pl.when(prefetch_pred)
                def _():
                    idx_copy(c + 2, q).start()

        def body(g, carry):
            for u in (0, 1):
                c = 2 * g + u
                p, q = u, 1 - u
                @pl.when(g >= 1)
                def _():
                    write(c - 2, p).wait()
                idx_copy(c, p).wait()
                for j in range(_CHUNK):
                    gather(c, p, j).start()
                if u == 0:
                    @pl.when(g >= 1)
                    def _():
                        retire(c - 1, q, True)
                else:
                    retire(c - 1, q, c + 2 < n_chunks)
            return carry

        lax.fori_loop(0, half, body, 0)

        last = n_chunks - 1
        for j in range(_CHUNK):
            gather(last, 1, j).wait()
        write(last, 1).start()
        write(last - 1, 0).wait()
        write(last, 1).wait()

    return lookup(table_proj, idx2d)


def kernel(x, embed_table, proj_w, proj_b):
    bsz, seq = x.shape
    d = embed_table.shape[1]
    table_proj = _project_table(embed_table, proj_w, proj_b)
    idx2d = x.reshape(-1, _GATHER).astype(jnp.int32)
    out_flat = _sc_lookup(table_proj, idx2d)
    return out_flat.reshape(bsz, seq, d)
